# baseline (speedup 1.0000x reference)
import functools

import jax
import jax.numpy as jnp
from jax import lax
from jax.experimental import pallas as pl
from jax.experimental.pallas import tpu as pltpu
from jax.experimental.pallas import tpu_sc as plsc

_B, _N, _E, _CIN, _H, _COUT, _NB = 2, 10000, 160000, 3, 256, 3, 6
_HH = _H // 2
_ET = _E + _N
_K = 128
_NCHUNK = 88
_EP = 16 * _NCHUNK * _K
_NPAD = 10240
_NSYM = 16384
_TRASH = 10100
_BN = 1000
_NBLK = _N // _BN

_f32 = jnp.float32
_mesh = plsc.VectorSubcoreMesh(
    core_axis_name="c", subcore_axis_name="s", num_cores=2, num_subcores=16)


def _sds(shape, dtype=_f32):
  return jax.ShapeDtypeStruct(shape, dtype)


def _agg128_body(ylo, yhi, col_r, row_r, symm_r, zsrc,
                 slo, shi, glo, ghi,
                 acc, colidx, symmidx, rbuf, dbuf, gsem, ssem, rsem):
  c = lax.axis_index("c")
  sid = lax.axis_index("s")
  nb = 2
  for b in range(_B):
    pltpu.sync_copy(zsrc, dbuf.at[0])
    for t in range(5):
      pltpu.sync_copy(dbuf.at[0], acc.at[pl.ds(sid * 640 + t * _K, _K)])
    plsc.subcore_barrier()
    pltpu.sync_copy(col_r.at[b].at[sid], colidx)
    rows = row_r.at[b].at[sid]
    for cv in range(2):
      ysrc = ylo if cv == 0 else yhi

      @pl.when(c == cv)
      def _(ysrc=ysrc, rows=rows):
        for t in range(nb):
          pltpu.async_copy(rows.at[t], rbuf.at[t], rsem.at[t])
          pltpu.async_copy(ysrc.at[colidx.at[t]], dbuf.at[t], gsem.at[t])

        def pipe(i, carry):
          jj = i * nb
          for t in range(nb):
            pltpu.make_async_copy(rows.at[jj + t], rbuf.at[t],
                                  rsem.at[t]).wait()
            pltpu.make_async_copy(
                ysrc.at[colidx.at[jj + t]], dbuf.at[t], gsem.at[t]).wait()
            pltpu.async_copy(dbuf.at[t], acc.at[rbuf.at[t]],
                             ssem.at[t], add=True)
          for t in range(nb):
            pltpu.make_async_copy(dbuf.at[t], acc.at[rbuf.at[t]],
                                  ssem.at[t]).wait()
            pltpu.async_copy(rows.at[jj + nb + t], rbuf.at[t], rsem.at[t])
            pltpu.async_copy(ysrc.at[colidx.at[jj + nb + t]], dbuf.at[t],
                             gsem.at[t])
          return carry
        lax.fori_loop(0, _NCHUNK // nb - 1, pipe, 0)
        jj = _NCHUNK - nb
        for t in range(nb):
          pltpu.make_async_copy(rows.at[jj + t], rbuf.at[t], rsem.at[t]).wait()
          pltpu.make_async_copy(
              ysrc.at[colidx.at[jj + t]], dbuf.at[t], gsem.at[t]).wait()
          pltpu.async_copy(dbuf.at[t], acc.at[rbuf.at[t]],
                           ssem.at[t], add=True)
        for t in range(nb):
          pltpu.make_async_copy(dbuf.at[t], acc.at[rbuf.at[t]],
                                ssem.at[t]).wait()
    plsc.subcore_barrier()
    for cv in range(2):
      sdst = slo if cv == 0 else shi

      @pl.when(c == cv)
      def _(sdst=sdst, b=b):
        pltpu.sync_copy(acc.at[pl.ds(sid * 640, 640)],
                        sdst.at[b].at[pl.ds(sid * 640, 640)])
    pltpu.sync_copy(symm_r.at[b].at[sid], symmidx)
    for cv in range(2):
      gdst = glo if cv == 0 else ghi

      @pl.when(c == cv)
      def _(gdst=gdst, b=b):
        pltpu.async_copy(acc.at[symmidx.at[0]], dbuf.at[0], gsem.at[0])
        for j in range(8):
          pltpu.make_async_copy(
              acc.at[symmidx.at[j]], dbuf.at[j % 2], gsem.at[j % 2]).wait()
          if j < 7:
            pltpu.async_copy(acc.at[symmidx.at[j + 1]], dbuf.at[(j + 1) % 2],
                             gsem.at[(j + 1) % 2])
          pltpu.sync_copy(dbuf.at[j % 2],
                          gdst.at[b].at[pl.ds(sid * 1024 + j * _K, _K)])
    plsc.subcore_barrier()


_agg128 = functools.partial(
    pl.kernel, _agg128_body, mesh=_mesh,
    out_type=[_sds((_B, _NPAD, _HH)), _sds((_B, _NPAD, _HH)),
              _sds((_B, _NSYM, _HH)), _sds((_B, _NSYM, _HH))],
    scratch_types=[
        pltpu.VMEM_SHARED((_NPAD, _HH), _f32),
        pltpu.VMEM((_NCHUNK, _K), jnp.int32),
        pltpu.VMEM((8, _K), jnp.int32),
        pltpu.VMEM((2, _K), jnp.int32),
        pltpu.VMEM((2, _K, _HH), _f32),
        pltpu.SemaphoreType.DMA((2,)),
        pltpu.SemaphoreType.DMA((2,)),
        pltpu.SemaphoreType.DMA((2,)),
    ])()


def _dsym_body(dinvcat, symm2_r, dsym16, idxbuf, dbuf, sem):
  c = lax.axis_index("c")
  sid = lax.axis_index("s")
  for b in range(_B):
    @pl.when(c == b)
    def _(b=b):
      pltpu.sync_copy(symm2_r.at[b].at[sid], idxbuf)
      for j in range(8):
        pltpu.async_copy(dinvcat.at[idxbuf.at[j]], dbuf, sem).wait()
        pltpu.sync_copy(dbuf, dsym16.at[b].at[pl.ds(sid * 1024 + j * _K, _K)])


_dsym = functools.partial(
    pl.kernel, _dsym_body, mesh=_mesh,
    out_type=_sds((_B, _NSYM, _HH)),
    scratch_types=[
        pltpu.VMEM((8, _K), jnp.int32),
        pltpu.VMEM((_K, _HH), _f32),
        pltpu.SemaphoreType.DMA,
    ])()


def _row_spec(w):
  return pl.BlockSpec((1, _BN, w), lambda b, i: (b, i, 0))


def _full_spec(shape):
  return pl.BlockSpec(shape, lambda b, i: (0,) * len(shape))


def _tc_call(body, in_specs, out_specs, out_shape):
  return pl.pallas_call(body, grid=(_B, _NBLK), in_specs=in_specs,
                        out_specs=out_specs, out_shape=out_shape)


def _dinv_body(deg, o):
  o[...] = lax.rsqrt(deg[...])


def _k1_body(inp, dv, wl, blb, w1, ylo, yhi):
  u = jnp.dot(inp[0], wl[...], preferred_element_type=_f32) + blb[0:1, :]
  u = u * dv[0][:, 0:1]
  y = jnp.dot(u, w1[...], preferred_element_type=_f32)
  ylo[0] = y[:, :_HH]
  yhi[0] = y[:, _HH:]


def _k2_body(slo, shi, dv, b1b, w, xlo, xhi, ylo, yhi):
  d = dv[0][:, 0:1]
  xl = jnp.maximum(d * slo[0] + b1b[0:1, :_HH], 0.0)
  xh = jnp.maximum(d * shi[0] + b1b[0:1, _HH:], 0.0)
  xlo[0] = xl
  xhi[0] = xh
  u = jnp.concatenate([xl * d, xh * d], axis=1)
  y = jnp.dot(u, w[...], preferred_element_type=_f32)
  ylo[0] = y[:, :_HH]
  yhi[0] = y[:, _HH:]


def _k3_body(slo, shi, glo, ghi, dv, dsy, bb, w, ylo, yhi):
  d = dv[0][:, 0:1]
  dm = dsy[0][:, 0:1]
  tl = jnp.maximum(0.5 * (d * slo[0] + dm * glo[0]) + bb[0:1, :_HH], 0.0)
  th = jnp.maximum(0.5 * (d * shi[0] + dm * ghi[0]) + bb[0:1, _HH:], 0.0)
  u = jnp.concatenate([tl * d, th * d], axis=1)
  y = jnp.dot(u, w[...], preferred_element_type=_f32)
  ylo[0] = y[:, :_HH]
  yhi[0] = y[:, _HH:]


def _k4_body(wout, slo, shi, glo, ghi, x0lo, x0hi, dv, dsy, bb, w,
             xlo, xhi, *yout):
  d = dv[0][:, 0:1]
  dm = dsy[0][:, 0:1]
  tl = jnp.maximum(0.5 * (d * slo[0] + dm * glo[0]) + bb[0:1, :_HH], 0.0)
  th = jnp.maximum(0.5 * (d * shi[0] + dm * ghi[0]) + bb[0:1, _HH:], 0.0)
  xnl = 0.5 * (x0lo[0] + tl)
  xnh = 0.5 * (x0hi[0] + th)
  xlo[0] = xnl
  xhi[0] = xnh
  u = jnp.concatenate([xnl * d, xnh * d], axis=1)
  y = jnp.dot(u, w[...], preferred_element_type=_f32)
  if wout == _H:
    yout[0][0] = y[:, :_HH]
    yout[1][0] = y[:, _HH:]
  else:
    yout[0][0] = y


def _k5_body(s16, dv, b3b, o):
  o[0] = dv[0][:, 0:1] * s16[0] + b3b[0:1, :]


def _bcast(v):
  return jnp.broadcast_to(v.reshape(1, -1), (8, v.shape[-1]))


def kernel(input, edge, symm_update, form_batch, Wl, bl, W1, b1,
           block_W1, block_b1, block_W2, block_b2, W3, b3):
  del form_batch
  edge = edge.astype(jnp.int32)
  symm = symm_update.astype(jnp.int32)

  sl = jnp.arange(_N, dtype=jnp.int32)
  sl2 = jnp.broadcast_to(sl[None, :], (_B, _N))
  row = jnp.concatenate([edge[:, 0, :], sl2], axis=1)
  col = jnp.concatenate([edge[:, 1, :], sl2], axis=1)
  row = jnp.pad(row, ((0, 0), (0, _EP - _ET)), constant_values=_TRASH)
  col = jnp.pad(col, ((0, 0), (0, _EP - _ET)), constant_values=0)
  boff = (jnp.arange(_B, dtype=jnp.int32) * _NPAD)[:, None]
  comb = jnp.sort(row * 32768 + (col + boff), axis=1)
  col2_r = (comb % 32768).reshape(_B, 16, _NCHUNK, _K)
  row_r = (comb // 32768).reshape(_B, 16, _NCHUNK, _K)
  symm_p = jnp.pad(symm, ((0, 0), (0, _NSYM - _N)))
  symm_r = symm_p.reshape(_B, 16, 8, _K)
  symm2_r = (symm_p + boff).reshape(_B, 16, 8, _K)

  zv128 = jnp.zeros((_K, _HH), _f32)
  onescat = jnp.ones((_B * _NPAD, _HH), _f32)

  inp_pad = jnp.pad(input.astype(_f32), ((0, 0), (0, 0), (0, 128 - _CIN)))
  wlp = jnp.pad(Wl, ((0, 128 - _CIN), (0, 0)))
  w3p = jnp.pad(W3, ((0, 0), (0, _HH - _COUT)))
  b3b = _bcast(jnp.pad(b3, (0, _HH - _COUT)))
  blb = _bcast(bl)
  b1b = _bcast(b1)

  deg, _, _, _ = _agg128(onescat, onescat, col2_r, row_r, symm_r, zv128)
  rs = _row_spec(_HH)
  rs16 = rs
  dinv16 = _tc_call(_dinv_body, [rs], rs, _sds((_B, _NPAD, _HH)))(deg)
  dsym16 = _dsym(dinv16.reshape(_B * _NPAD, _HH), symm2_r)

  ylo, yhi = _tc_call(
      _k1_body,
      [_row_spec(128), rs16, _full_spec((128, _H)), _full_spec((8, _H)),
       _full_spec((_H, _H))],
      [rs, rs], [_sds((_B, _NPAD, _HH))] * 2,
  )(inp_pad, dinv16, wlp, blb, W1)

  slo, shi, _, _ = _agg128(ylo.reshape(-1, _HH), yhi.reshape(-1, _HH),
                           col2_r, row_r, symm_r, zv128)

  xlo, xhi, ylo, yhi = _tc_call(
      _k2_body,
      [rs, rs, rs16, _full_spec((8, _H)), _full_spec((_H, _H))],
      [rs] * 4, [_sds((_B, _NPAD, _HH))] * 4,
  )(slo, shi, dinv16, b1b, block_W1[0])

  for i in range(_NB):
    slo, shi, glo, ghi = _agg128(ylo.reshape(-1, _HH), yhi.reshape(-1, _HH),
                                 col2_r, row_r, symm_r, zv128)
    ylo, yhi = _tc_call(
        _k3_body,
        [rs, rs, rs, rs, rs16, rs16, _full_spec((8, _H)),
         _full_spec((_H, _H))],
        [rs] * 2, [_sds((_B, _NPAD, _HH))] * 2,
    )(slo, shi, glo, ghi, dinv16, dsym16, _bcast(block_b1[i]), block_W2[i])

    slo, shi, glo, ghi = _agg128(ylo.reshape(-1, _HH), yhi.reshape(-1, _HH),
                                 col2_r, row_r, symm_r, zv128)
    if i < _NB - 1:
      xlo, xhi, ylo, yhi = _tc_call(
          functools.partial(_k4_body, _H),
          [rs, rs, rs, rs, rs, rs, rs16, rs16, _full_spec((8, _H)),
           _full_spec((_H, _H))],
          [rs] * 4, [_sds((_B, _NPAD, _HH))] * 4,
      )(slo, shi, glo, ghi, xlo, xhi, dinv16, dsym16,
        _bcast(block_b2[i]), block_W1[i + 1])
    else:
      xlo, xhi, y16 = _tc_call(
          functools.partial(_k4_body, _HH),
          [rs, rs, rs, rs, rs, rs, rs16, rs16, _full_spec((8, _H)),
           _full_spec((_H, _HH))],
          [rs, rs, rs],
          [_sds((_B, _NPAD, _HH))] * 3,
      )(slo, shi, glo, ghi, xlo, xhi, dinv16, dsym16,
        _bcast(block_b2[i]), w3p)

  y16c = y16.reshape(-1, _HH)
  s16, _, _, _ = _agg128(y16c, y16c, col2_r, row_r, symm_r, zv128)
  o16 = _tc_call(
      _k5_body, [rs, rs, _full_spec((8, _HH))], rs, _sds((_B, _NPAD, _HH)),
  )(s16, dinv16, b3b)

  xs = jnp.concatenate([xlo, xhi], axis=2)[:, :_N]
  outs = o16[:, :_N, :_COUT]
  return (xs, outs)


# speedup vs baseline: 1.0057x; 1.0057x over previous
import functools

import jax
import jax.numpy as jnp
from jax import lax
from jax.experimental import pallas as pl
from jax.experimental.pallas import tpu as pltpu
from jax.experimental.pallas import tpu_sc as plsc

_B, _N, _E, _CIN, _H, _COUT, _NB = 2, 10000, 160000, 3, 256, 3, 6
_HH = _H // 2
_ET = _E + _N
_K = 128
_NCHUNK = 88
_EP = 16 * _NCHUNK * _K
_NPAD = 10240
_NSYM = 16384
_TRASH = 10100
_BN = 1000
_NBLK = _N // _BN

_f32 = jnp.float32
_mesh = plsc.VectorSubcoreMesh(
    core_axis_name="c", subcore_axis_name="s", num_cores=2, num_subcores=16)


def _sds(shape, dtype=_f32):
  return jax.ShapeDtypeStruct(shape, dtype)


def _agg128_body(ylo, yhi, col_r, row_r, symm_r, zsrc,
                 slo, shi, glo, ghi,
                 acc, colidx, symmidx, rbuf, dbuf, gsem, ssem, rsem):
  c = lax.axis_index("c")
  sid = lax.axis_index("s")
  nb = 2
  for b in range(_B):
    pltpu.sync_copy(zsrc, dbuf.at[0])
    for t in range(5):
      pltpu.sync_copy(dbuf.at[0], acc.at[pl.ds(sid * 640 + t * _K, _K)])
    plsc.subcore_barrier()
    pltpu.sync_copy(col_r.at[b].at[sid], colidx)
    rows = row_r.at[b].at[sid]
    for cv in range(2):
      ysrc = ylo if cv == 0 else yhi

      @pl.when(c == cv)
      def _(ysrc=ysrc, rows=rows):
        for t in range(nb):
          pltpu.async_copy(rows.at[t], rbuf.at[t], rsem.at[t])
          pltpu.async_copy(ysrc.at[colidx.at[t]], dbuf.at[t], gsem.at[t])

        def pipe(i, carry):
          jj = i * nb
          for t in range(nb):
            pltpu.make_async_copy(rows.at[jj + t], rbuf.at[t],
                                  rsem.at[t]).wait()
            pltpu.make_async_copy(
                ysrc.at[colidx.at[jj + t]], dbuf.at[t], gsem.at[t]).wait()
            pltpu.async_copy(dbuf.at[t], acc.at[rbuf.at[t]],
                             ssem.at[t], add=True)
          for t in range(nb):
            pltpu.make_async_copy(dbuf.at[t], acc.at[rbuf.at[t]],
                                  ssem.at[t]).wait()
            pltpu.async_copy(rows.at[jj + nb + t], rbuf.at[t], rsem.at[t])
            pltpu.async_copy(ysrc.at[colidx.at[jj + nb + t]], dbuf.at[t],
                             gsem.at[t])
          return carry
        lax.fori_loop(0, _NCHUNK // nb - 1, pipe, 0)
        jj = _NCHUNK - nb
        for t in range(nb):
          pltpu.make_async_copy(rows.at[jj + t], rbuf.at[t], rsem.at[t]).wait()
          pltpu.make_async_copy(
              ysrc.at[colidx.at[jj + t]], dbuf.at[t], gsem.at[t]).wait()
          pltpu.async_copy(dbuf.at[t], acc.at[rbuf.at[t]],
                           ssem.at[t], add=True)
        for t in range(nb):
          pltpu.make_async_copy(dbuf.at[t], acc.at[rbuf.at[t]],
                                ssem.at[t]).wait()
    plsc.subcore_barrier()
    for cv in range(2):
      sdst = slo if cv == 0 else shi

      @pl.when(c == cv)
      def _(sdst=sdst, b=b):
        pltpu.sync_copy(acc.at[pl.ds(sid * 640, 640)],
                        sdst.at[b].at[pl.ds(sid * 640, 640)])
    pltpu.sync_copy(symm_r.at[b].at[sid], symmidx)
    for cv in range(2):
      gdst = glo if cv == 0 else ghi

      @pl.when(c == cv)
      def _(gdst=gdst, b=b):
        pltpu.async_copy(acc.at[symmidx.at[0]], dbuf.at[0], gsem.at[0])
        for j in range(8):
          pltpu.make_async_copy(
              acc.at[symmidx.at[j]], dbuf.at[j % 2], gsem.at[j % 2]).wait()
          if j < 7:
            pltpu.async_copy(acc.at[symmidx.at[j + 1]], dbuf.at[(j + 1) % 2],
                             gsem.at[(j + 1) % 2])
          pltpu.sync_copy(dbuf.at[j % 2],
                          gdst.at[b].at[pl.ds(sid * 1024 + j * _K, _K)])
    plsc.subcore_barrier()


_agg128 = functools.partial(
    pl.kernel, _agg128_body, mesh=_mesh,
    out_type=[_sds((_B, _NPAD, _HH)), _sds((_B, _NPAD, _HH)),
              _sds((_B, _NSYM, _HH)), _sds((_B, _NSYM, _HH))],
    scratch_types=[
        pltpu.VMEM_SHARED((_NPAD, _HH), _f32),
        pltpu.VMEM((_NCHUNK, _K), jnp.int32),
        pltpu.VMEM((8, _K), jnp.int32),
        pltpu.VMEM((2, _K), jnp.int32),
        pltpu.VMEM((2, _K, _HH), _f32),
        pltpu.SemaphoreType.DMA((2,)),
        pltpu.SemaphoreType.DMA((2,)),
        pltpu.SemaphoreType.DMA((2,)),
    ])()


def _dsym_body(dinvcat, symm2_r, dsym16, idxbuf, dbuf, sem):
  c = lax.axis_index("c")
  sid = lax.axis_index("s")
  for b in range(_B):
    @pl.when(c == b)
    def _(b=b):
      pltpu.sync_copy(symm2_r.at[b].at[sid], idxbuf)
      for j in range(8):
        pltpu.async_copy(dinvcat.at[idxbuf.at[j]], dbuf, sem).wait()
        pltpu.sync_copy(dbuf, dsym16.at[b].at[pl.ds(sid * 1024 + j * _K, _K)])


_dsym = functools.partial(
    pl.kernel, _dsym_body, mesh=_mesh,
    out_type=_sds((_B, _NSYM, _HH)),
    scratch_types=[
        pltpu.VMEM((8, _K), jnp.int32),
        pltpu.VMEM((_K, _HH), _f32),
        pltpu.SemaphoreType.DMA,
    ])()


def _row_spec(w):
  return pl.BlockSpec((1, _BN, w), lambda b, i: (b, i, 0))


def _full_spec(shape):
  return pl.BlockSpec(shape, lambda b, i: (0,) * len(shape))


def _tc_call(body, in_specs, out_specs, out_shape):
  return pl.pallas_call(body, grid=(_B, _NBLK), in_specs=in_specs,
                        out_specs=out_specs, out_shape=out_shape)


def _dinv_body(deg, o):
  o[...] = lax.rsqrt(deg[...])


def _k1_body(inp, dv, wl, blb, w1, ylo, yhi):
  u = jnp.dot(inp[0], wl[...], preferred_element_type=_f32) + blb[0:1, :]
  u = u * dv[0][:, 0:1]
  y = jnp.dot(u, w1[...], preferred_element_type=_f32)
  ylo[0] = y[:, :_HH]
  yhi[0] = y[:, _HH:]


def _k2_body(slo, shi, dv, b1b, w, xlo, xhi, ylo, yhi):
  d = dv[0][:, 0:1]
  xl = jnp.maximum(d * slo[0] + b1b[0:1, :_HH], 0.0)
  xh = jnp.maximum(d * shi[0] + b1b[0:1, _HH:], 0.0)
  xlo[0] = xl
  xhi[0] = xh
  u = jnp.concatenate([xl * d, xh * d], axis=1)
  y = jnp.dot(u, w[...], preferred_element_type=_f32)
  ylo[0] = y[:, :_HH]
  yhi[0] = y[:, _HH:]


def _k3_body(slo, shi, glo, ghi, dv, dsy, bb, w, ylo, yhi):
  d = dv[0][:, 0:1]
  dm = dsy[0][:, 0:1]
  tl = jnp.maximum(0.5 * (d * slo[0] + dm * glo[0]) + bb[0:1, :_HH], 0.0)
  th = jnp.maximum(0.5 * (d * shi[0] + dm * ghi[0]) + bb[0:1, _HH:], 0.0)
  u = jnp.concatenate([tl * d, th * d], axis=1)
  y = jnp.dot(u, w[...], preferred_element_type=_f32)
  ylo[0] = y[:, :_HH]
  yhi[0] = y[:, _HH:]


def _k4_body(wout, slo, shi, glo, ghi, x0lo, x0hi, dv, dsy, bb, w,
             xlo, xhi, *yout):
  d = dv[0][:, 0:1]
  dm = dsy[0][:, 0:1]
  tl = jnp.maximum(0.5 * (d * slo[0] + dm * glo[0]) + bb[0:1, :_HH], 0.0)
  th = jnp.maximum(0.5 * (d * shi[0] + dm * ghi[0]) + bb[0:1, _HH:], 0.0)
  xnl = 0.5 * (x0lo[0] + tl)
  xnh = 0.5 * (x0hi[0] + th)
  xlo[0] = xnl
  xhi[0] = xnh
  u = jnp.concatenate([xnl * d, xnh * d], axis=1)
  y = jnp.dot(u, w[...], preferred_element_type=_f32)
  if wout == _H:
    yout[0][0] = y[:, :_HH]
    yout[1][0] = y[:, _HH:]
  else:
    yout[0][0] = y


def _k5_body(s16, dv, b3b, o):
  o[0] = dv[0][:, 0:1] * s16[0] + b3b[0:1, :]


def _bcast(v):
  return jnp.broadcast_to(v.reshape(1, -1), (8, v.shape[-1]))


def kernel(input, edge, symm_update, form_batch, Wl, bl, W1, b1,
           block_W1, block_b1, block_W2, block_b2, W3, b3):
  del form_batch
  edge = edge.astype(jnp.int32)
  symm = symm_update.astype(jnp.int32)

  sl = jnp.arange(_N, dtype=jnp.int32)
  sl2 = jnp.broadcast_to(sl[None, :], (_B, _N))
  row = jnp.concatenate([edge[:, 0, :], sl2], axis=1)
  col = jnp.concatenate([edge[:, 1, :], sl2], axis=1)
  row = jnp.pad(row, ((0, 0), (0, _EP - _ET)), constant_values=_TRASH)
  col = jnp.pad(col, ((0, 0), (0, _EP - _ET)), constant_values=0)
  boff = (jnp.arange(_B, dtype=jnp.int32) * _NPAD)[:, None]
  comb = jnp.sort((col + boff) * 16384 + row, axis=1)
  col2_r = (comb // 16384).reshape(_B, 16, _NCHUNK, _K)
  row_r = (comb % 16384).reshape(_B, 16, _NCHUNK, _K)
  symm_p = jnp.pad(symm, ((0, 0), (0, _NSYM - _N)))
  symm_r = symm_p.reshape(_B, 16, 8, _K)
  symm2_r = (symm_p + boff).reshape(_B, 16, 8, _K)

  zv128 = jnp.zeros((_K, _HH), _f32)
  onescat = jnp.ones((_B * _NPAD, _HH), _f32)

  inp_pad = jnp.pad(input.astype(_f32), ((0, 0), (0, 0), (0, 128 - _CIN)))
  wlp = jnp.pad(Wl, ((0, 128 - _CIN), (0, 0)))
  w3p = jnp.pad(W3, ((0, 0), (0, _HH - _COUT)))
  b3b = _bcast(jnp.pad(b3, (0, _HH - _COUT)))
  blb = _bcast(bl)
  b1b = _bcast(b1)

  deg, _, _, _ = _agg128(onescat, onescat, col2_r, row_r, symm_r, zv128)
  rs = _row_spec(_HH)
  rs16 = rs
  dinv16 = _tc_call(_dinv_body, [rs], rs, _sds((_B, _NPAD, _HH)))(deg)
  dsym16 = _dsym(dinv16.reshape(_B * _NPAD, _HH), symm2_r)

  ylo, yhi = _tc_call(
      _k1_body,
      [_row_spec(128), rs16, _full_spec((128, _H)), _full_spec((8, _H)),
       _full_spec((_H, _H))],
      [rs, rs], [_sds((_B, _NPAD, _HH))] * 2,
  )(inp_pad, dinv16, wlp, blb, W1)

  slo, shi, _, _ = _agg128(ylo.reshape(-1, _HH), yhi.reshape(-1, _HH),
                           col2_r, row_r, symm_r, zv128)

  xlo, xhi, ylo, yhi = _tc_call(
      _k2_body,
      [rs, rs, rs16, _full_spec((8, _H)), _full_spec((_H, _H))],
      [rs] * 4, [_sds((_B, _NPAD, _HH))] * 4,
  )(slo, shi, dinv16, b1b, block_W1[0])

  for i in range(_NB):
    slo, shi, glo, ghi = _agg128(ylo.reshape(-1, _HH), yhi.reshape(-1, _HH),
                                 col2_r, row_r, symm_r, zv128)
    ylo, yhi = _tc_call(
        _k3_body,
        [rs, rs, rs, rs, rs16, rs16, _full_spec((8, _H)),
         _full_spec((_H, _H))],
        [rs] * 2, [_sds((_B, _NPAD, _HH))] * 2,
    )(slo, shi, glo, ghi, dinv16, dsym16, _bcast(block_b1[i]), block_W2[i])

    slo, shi, glo, ghi = _agg128(ylo.reshape(-1, _HH), yhi.reshape(-1, _HH),
                                 col2_r, row_r, symm_r, zv128)
    if i < _NB - 1:
      xlo, xhi, ylo, yhi = _tc_call(
          functools.partial(_k4_body, _H),
          [rs, rs, rs, rs, rs, rs, rs16, rs16, _full_spec((8, _H)),
           _full_spec((_H, _H))],
          [rs] * 4, [_sds((_B, _NPAD, _HH))] * 4,
      )(slo, shi, glo, ghi, xlo, xhi, dinv16, dsym16,
        _bcast(block_b2[i]), block_W1[i + 1])
    else:
      xlo, xhi, y16 = _tc_call(
          functools.partial(_k4_body, _HH),
          [rs, rs, rs, rs, rs, rs, rs16, rs16, _full_spec((8, _H)),
           _full_spec((_H, _HH))],
          [rs, rs, rs],
          [_sds((_B, _NPAD, _HH))] * 3,
      )(slo, shi, glo, ghi, xlo, xhi, dinv16, dsym16,
        _bcast(block_b2[i]), w3p)

  y16c = y16.reshape(-1, _HH)
  s16, _, _, _ = _agg128(y16c, y16c, col2_r, row_r, symm_r, zv128)
  o16 = _tc_call(
      _k5_body, [rs, rs, _full_spec((8, _HH))], rs, _sds((_B, _NPAD, _HH)),
  )(s16, dinv16, b3b)

  xs = jnp.concatenate([xlo, xhi], axis=2)[:, :_N]
  outs = o16[:, :_N, :_COUT]
  return (xs, outs)


# speedup vs baseline: 1.5414x; 1.5326x over previous
import functools

import jax
import jax.numpy as jnp
from jax import lax
from jax.experimental import pallas as pl
from jax.experimental.pallas import tpu as pltpu
from jax.experimental.pallas import tpu_sc as plsc

_B, _N, _E, _CIN, _H, _COUT, _NB = 2, 10000, 160000, 3, 256, 3, 6
_HH = _H // 2
_ET = _E + _N
_K = 128
_NCHUNK = 88
_EP = 16 * _NCHUNK * _K
_NPAD = 10240
_NSYM = 16384
_NW = 80
_WPT = 5
_WCAP = 2560
_WCH = _WCAP // 64
_TRASH = 10100
_BN = 1000
_NBLK = _N // _BN

_f32 = jnp.float32
_mesh = plsc.VectorSubcoreMesh(
    core_axis_name="c", subcore_axis_name="s", num_cores=2, num_subcores=16)


def _sds(shape, dtype=_f32):
  return jax.ShapeDtypeStruct(shape, dtype)


def _agg128_body(ylo, yhi, colloc_r, roww_r, symm_r, zsrc,
                 slo, shi, glo, ghi,
                 acc, ywin_sh, cbuf, rbuf, symmidx, dbuf, msem, ssem):
  c = lax.axis_index("c")
  sid = lax.axis_index("s")
  for b in range(_B):
    pltpu.sync_copy(zsrc, dbuf.at[0])
    for t in range(10):
      pltpu.sync_copy(dbuf.at[0], acc.at[pl.ds(sid * 640 + t * 64, 64)])
    plsc.subcore_barrier()
    for cv in range(2):
      ysrc = ylo if cv == 0 else yhi

      @pl.when(c == cv)
      def _(ysrc=ysrc, b=b):
        for wi in range(_WPT):
          w = sid * _WPT + wi
          pltpu.sync_copy(ysrc.at[pl.ds(b * _NPAD + w * 128, 128)],
                          ywin_sh.at[sid])
          pltpu.sync_copy(colloc_r.at[b].at[w], cbuf)
          pltpu.sync_copy(roww_r.at[b].at[w], rbuf)
          for t in range(2):
            pltpu.async_copy(ywin_sh.at[sid].at[cbuf.at[t]], dbuf.at[t],
                             msem.at[t])

          def pipe(i, carry):
            jj = i * 2
            for t in range(2):
              pltpu.make_async_copy(ywin_sh.at[sid].at[cbuf.at[jj + t]],
                                    dbuf.at[t], msem.at[t]).wait()
              pltpu.async_copy(dbuf.at[t], acc.at[rbuf.at[jj + t]],
                               ssem.at[t], add=True)
            for t in range(2):
              pltpu.make_async_copy(dbuf.at[t], acc.at[rbuf.at[jj + t]],
                                    ssem.at[t]).wait()
              pltpu.async_copy(ywin_sh.at[sid].at[cbuf.at[jj + 2 + t]],
                               dbuf.at[t], msem.at[t])
            return carry
          lax.fori_loop(0, _WCH // 2 - 1, pipe, 0)
          jj = _WCH - 2
          for t in range(2):
            pltpu.make_async_copy(ywin_sh.at[sid].at[cbuf.at[jj + t]],
                                  dbuf.at[t], msem.at[t]).wait()
            pltpu.async_copy(dbuf.at[t], acc.at[rbuf.at[jj + t]],
                             ssem.at[t], add=True)
          for t in range(2):
            pltpu.make_async_copy(dbuf.at[t], acc.at[rbuf.at[jj + t]],
                                  ssem.at[t]).wait()
    plsc.subcore_barrier()
    for cv in range(2):
      sdst = slo if cv == 0 else shi

      @pl.when(c == cv)
      def _(sdst=sdst, b=b):
        pltpu.sync_copy(acc.at[pl.ds(sid * 640, 640)],
                        sdst.at[b].at[pl.ds(sid * 640, 640)])
    pltpu.sync_copy(symm_r.at[b].at[sid], symmidx)
    for cv in range(2):
      gdst = glo if cv == 0 else ghi

      @pl.when(c == cv)
      def _(gdst=gdst, b=b):
        pltpu.async_copy(acc.at[symmidx.at[0]], dbuf.at[0], msem.at[0])
        for j in range(16):
          pltpu.make_async_copy(
              acc.at[symmidx.at[j]], dbuf.at[j % 2], msem.at[j % 2]).wait()
          if j < 15:
            pltpu.async_copy(acc.at[symmidx.at[j + 1]], dbuf.at[(j + 1) % 2],
                             msem.at[(j + 1) % 2])
          pltpu.sync_copy(dbuf.at[j % 2],
                          gdst.at[b].at[pl.ds(sid * 1024 + j * 64, 64)])
    plsc.subcore_barrier()


_agg128 = functools.partial(
    pl.kernel, _agg128_body, mesh=_mesh,
    out_type=[_sds((_B, _NPAD, _HH)), _sds((_B, _NPAD, _HH)),
              _sds((_B, _NSYM, _HH)), _sds((_B, _NSYM, _HH))],
    scratch_types=[
        pltpu.VMEM_SHARED((_NPAD, _HH), _f32),
        pltpu.VMEM_SHARED((16, 128, _HH), _f32),
        pltpu.VMEM((_WCH, 64), jnp.int32),
        pltpu.VMEM((_WCH, 64), jnp.int32),
        pltpu.VMEM((16, 64), jnp.int32),
        pltpu.VMEM((2, 64, _HH), _f32),
        pltpu.SemaphoreType.DMA((2,)),
        pltpu.SemaphoreType.DMA((2,)),
    ])()


def _dsym_body(dinvcat, symm2_r, dsym16, idxbuf, dbuf, sem):
  c = lax.axis_index("c")
  sid = lax.axis_index("s")
  for b in range(_B):
    @pl.when(c == b)
    def _(b=b):
      pltpu.sync_copy(symm2_r.at[b].at[sid], idxbuf)
      for j in range(8):
        pltpu.async_copy(dinvcat.at[idxbuf.at[j]], dbuf, sem).wait()
        pltpu.sync_copy(dbuf, dsym16.at[b].at[pl.ds(sid * 1024 + j * _K, _K)])


_dsym = functools.partial(
    pl.kernel, _dsym_body, mesh=_mesh,
    out_type=_sds((_B, _NSYM, _HH)),
    scratch_types=[
        pltpu.VMEM((8, _K), jnp.int32),
        pltpu.VMEM((_K, _HH), _f32),
        pltpu.SemaphoreType.DMA,
    ])()


def _row_spec(w):
  return pl.BlockSpec((1, _BN, w), lambda b, i: (b, i, 0))


def _full_spec(shape):
  return pl.BlockSpec(shape, lambda b, i: (0,) * len(shape))


def _tc_call(body, in_specs, out_specs, out_shape):
  return pl.pallas_call(body, grid=(_B, _NBLK), in_specs=in_specs,
                        out_specs=out_specs, out_shape=out_shape)


def _dinv_body(deg, o):
  o[...] = lax.rsqrt(deg[...])


def _k1_body(inp, dv, wl, blb, w1, ylo, yhi):
  u = jnp.dot(inp[0], wl[...], preferred_element_type=_f32) + blb[0:1, :]
  u = u * dv[0][:, 0:1]
  y = jnp.dot(u, w1[...], preferred_element_type=_f32)
  ylo[0] = y[:, :_HH]
  yhi[0] = y[:, _HH:]


def _k2_body(slo, shi, dv, b1b, w, xlo, xhi, ylo, yhi):
  d = dv[0][:, 0:1]
  xl = jnp.maximum(d * slo[0] + b1b[0:1, :_HH], 0.0)
  xh = jnp.maximum(d * shi[0] + b1b[0:1, _HH:], 0.0)
  xlo[0] = xl
  xhi[0] = xh
  u = jnp.concatenate([xl * d, xh * d], axis=1)
  y = jnp.dot(u, w[...], preferred_element_type=_f32)
  ylo[0] = y[:, :_HH]
  yhi[0] = y[:, _HH:]


def _k3_body(slo, shi, glo, ghi, dv, dsy, bb, w, ylo, yhi):
  d = dv[0][:, 0:1]
  dm = dsy[0][:, 0:1]
  tl = jnp.maximum(0.5 * (d * slo[0] + dm * glo[0]) + bb[0:1, :_HH], 0.0)
  th = jnp.maximum(0.5 * (d * shi[0] + dm * ghi[0]) + bb[0:1, _HH:], 0.0)
  u = jnp.concatenate([tl * d, th * d], axis=1)
  y = jnp.dot(u, w[...], preferred_element_type=_f32)
  ylo[0] = y[:, :_HH]
  yhi[0] = y[:, _HH:]


def _k4_body(wout, slo, shi, glo, ghi, x0lo, x0hi, dv, dsy, bb, w,
             xlo, xhi, *yout):
  d = dv[0][:, 0:1]
  dm = dsy[0][:, 0:1]
  tl = jnp.maximum(0.5 * (d * slo[0] + dm * glo[0]) + bb[0:1, :_HH], 0.0)
  th = jnp.maximum(0.5 * (d * shi[0] + dm * ghi[0]) + bb[0:1, _HH:], 0.0)
  xnl = 0.5 * (x0lo[0] + tl)
  xnh = 0.5 * (x0hi[0] + th)
  xlo[0] = xnl
  xhi[0] = xnh
  u = jnp.concatenate([xnl * d, xnh * d], axis=1)
  y = jnp.dot(u, w[...], preferred_element_type=_f32)
  if wout == _H:
    yout[0][0] = y[:, :_HH]
    yout[1][0] = y[:, _HH:]
  else:
    yout[0][0] = y


def _k5_body(s16, dv, b3b, o):
  o[0] = dv[0][:, 0:1] * s16[0] + b3b[0:1, :]


def _bcast(v):
  return jnp.broadcast_to(v.reshape(1, -1), (8, v.shape[-1]))


def kernel(input, edge, symm_update, form_batch, Wl, bl, W1, b1,
           block_W1, block_b1, block_W2, block_b2, W3, b3):
  del form_batch
  edge = edge.astype(jnp.int32)
  symm = symm_update.astype(jnp.int32)

  sl = jnp.arange(_N, dtype=jnp.int32)
  sl2 = jnp.broadcast_to(sl[None, :], (_B, _N))
  row = jnp.concatenate([edge[:, 0, :], sl2], axis=1)
  col = jnp.concatenate([edge[:, 1, :], sl2], axis=1)
  boff = (jnp.arange(_B, dtype=jnp.int32) * _NPAD)[:, None]
  comb = jnp.sort((col + boff) * 16384 + row, axis=1)
  col2s = comb // 16384
  rows_s = (comb % 16384).astype(jnp.int32)
  wloc = (col2s - boff) // 128
  colloc = (col2s % 128).astype(jnp.int32)
  ca_list, ra_list = [], []
  ar = jnp.arange(_ET, dtype=jnp.int32)
  for b in range(_B):
    start = jnp.searchsorted(wloc[b], jnp.arange(_NW, dtype=wloc.dtype))
    rank = ar - start.astype(jnp.int32)[wloc[b]]
    dest = wloc[b].astype(jnp.int32) * _WCAP + rank
    ca_list.append(jnp.zeros((_NW * _WCAP,), jnp.int32)
                   .at[dest].set(colloc[b], mode="drop"))
    ra_list.append(jnp.full((_NW * _WCAP,), _TRASH, jnp.int32)
                   .at[dest].set(rows_s[b], mode="drop"))
  colloc_r = jnp.stack(ca_list).reshape(_B, _NW, _WCH, 64)
  roww_r = jnp.stack(ra_list).reshape(_B, _NW, _WCH, 64)
  symm_p = jnp.pad(symm, ((0, 0), (0, _NSYM - _N)))
  symm_r = symm_p.reshape(_B, 16, 16, 64)
  symm2_r = (symm_p + boff).reshape(_B, 16, 8, _K)

  zv128 = jnp.zeros((64, _HH), _f32)
  onescat = jnp.ones((_B * _NPAD, _HH), _f32)

  inp_pad = jnp.pad(input.astype(_f32), ((0, 0), (0, 0), (0, 128 - _CIN)))
  wlp = jnp.pad(Wl, ((0, 128 - _CIN), (0, 0)))
  w3p = jnp.pad(W3, ((0, 0), (0, _HH - _COUT)))
  b3b = _bcast(jnp.pad(b3, (0, _HH - _COUT)))
  blb = _bcast(bl)
  b1b = _bcast(b1)

  deg, _, _, _ = _agg128(onescat, onescat, colloc_r, roww_r, symm_r, zv128)
  rs = _row_spec(_HH)
  rs16 = rs
  dinv16 = _tc_call(_dinv_body, [rs], rs, _sds((_B, _NPAD, _HH)))(deg)
  dsym16 = _dsym(dinv16.reshape(_B * _NPAD, _HH), symm2_r)

  ylo, yhi = _tc_call(
      _k1_body,
      [_row_spec(128), rs16, _full_spec((128, _H)), _full_spec((8, _H)),
       _full_spec((_H, _H))],
      [rs, rs], [_sds((_B, _NPAD, _HH))] * 2,
  )(inp_pad, dinv16, wlp, blb, W1)

  slo, shi, _, _ = _agg128(ylo.reshape(-1, _HH), yhi.reshape(-1, _HH),
                           colloc_r, roww_r, symm_r, zv128)

  xlo, xhi, ylo, yhi = _tc_call(
      _k2_body,
      [rs, rs, rs16, _full_spec((8, _H)), _full_spec((_H, _H))],
      [rs] * 4, [_sds((_B, _NPAD, _HH))] * 4,
  )(slo, shi, dinv16, b1b, block_W1[0])

  for i in range(_NB):
    slo, shi, glo, ghi = _agg128(ylo.reshape(-1, _HH), yhi.reshape(-1, _HH),
                                 colloc_r, roww_r, symm_r, zv128)
    ylo, yhi = _tc_call(
        _k3_body,
        [rs, rs, rs, rs, rs16, rs16, _full_spec((8, _H)),
         _full_spec((_H, _H))],
        [rs] * 2, [_sds((_B, _NPAD, _HH))] * 2,
    )(slo, shi, glo, ghi, dinv16, dsym16, _bcast(block_b1[i]), block_W2[i])

    slo, shi, glo, ghi = _agg128(ylo.reshape(-1, _HH), yhi.reshape(-1, _HH),
                                 colloc_r, roww_r, symm_r, zv128)
    if i < _NB - 1:
      xlo, xhi, ylo, yhi = _tc_call(
          functools.partial(_k4_body, _H),
          [rs, rs, rs, rs, rs, rs, rs16, rs16, _full_spec((8, _H)),
           _full_spec((_H, _H))],
          [rs] * 4, [_sds((_B, _NPAD, _HH))] * 4,
      )(slo, shi, glo, ghi, xlo, xhi, dinv16, dsym16,
        _bcast(block_b2[i]), block_W1[i + 1])
    else:
      xlo, xhi, y16 = _tc_call(
          functools.partial(_k4_body, _HH),
          [rs, rs, rs, rs, rs, rs, rs16, rs16, _full_spec((8, _H)),
           _full_spec((_H, _HH))],
          [rs, rs, rs],
          [_sds((_B, _NPAD, _HH))] * 3,
      )(slo, shi, glo, ghi, xlo, xhi, dinv16, dsym16,
        _bcast(block_b2[i]), w3p)

  y16c = y16.reshape(-1, _HH)
  s16, _, _, _ = _agg128(y16c, y16c, colloc_r, roww_r, symm_r, zv128)
  o16 = _tc_call(
      _k5_body, [rs, rs, _full_spec((8, _HH))], rs, _sds((_B, _NPAD, _HH)),
  )(s16, dinv16, b3b)

  xs = jnp.concatenate([xlo, xhi], axis=2)[:, :_N]
  outs = o16[:, :_N, :_COUT]
  return (xs, outs)
